# Initial kernel scaffold; baseline (speedup 1.0000x reference)
import jax
import jax.numpy as jnp
from jax.experimental import pallas as pl


def kernel(x, edge_index, edge_attr, params):
    raise NotImplementedError("write your pallas kernel here")


# speedup vs baseline: 1.8165x; 1.8165x over previous
import functools

import jax
import jax.numpy as jnp
from jax import lax
from jax.experimental import pallas as pl
from jax.experimental.pallas import tpu as pltpu
from jax.experimental.pallas import tpu_sc as plsc

N = 10000
E = 320000
EMB = 64
EDGE_DIM = 4

NC = 2
NS = 16
NW = NC * NS
EPW = E // NW
SUB = 100
NSUB = 2
CHUNK = SUB * NSUB
NCHUNK = EPW // CHUNK
DROWS = E // CHUNK
SUB2 = 80
NCH2 = EPW // SUB2
NPAD = 10240
RPT = NPAD // NS
ZROWS = 128
EPS = 1e-5

_F32 = jnp.float32
_DOT = dict(preferred_element_type=jnp.float32, precision=lax.Precision.HIGHEST)

_vmesh = plsc.VectorSubcoreMesh(core_axis_name="c", subcore_axis_name="s")


def _dot(a, b):
    return lax.dot_general(a, b, (((1,), (0,)), ((), ())), **_DOT)


def _lin_body(x_ref, w_ref, b_ref, o_ref):
    o_ref[...] = _dot(x_ref[...], w_ref[...]) + b_ref[...]


def _lin(x, w, b):
    n = x.shape[0]
    fout = w.shape[1]
    return pl.pallas_call(
        _lin_body,
        out_shape=jax.ShapeDtypeStruct((n, fout), _F32),
    )(x, w, b.reshape(1, fout))


def _pq_body(h_ref, wab_ref, t_ref):
    h = h_ref[...]
    t_ref[0:N, :] = _dot(h, wab_ref[...])
    t_ref[N:NPAD, :] = jnp.zeros((NPAD - N, 2 * EMB), _F32)


def _pq(h, wab):
    return pl.pallas_call(
        _pq_body,
        out_shape=jax.ShapeDtypeStruct((NPAD, 2 * EMB), _F32),
    )(h, wab)


_RBLK = 4000


def _r_body(ea_ref, w_ref, b_ref, o_ref):
    ea = ea_ref[...]
    acc = b_ref[...]
    for k in range(EDGE_DIM):
        acc = acc + ea[:, k:k + 1] * w_ref[k:k + 1, :]
    o_ref[...] = acc


def _r(ea, w, b):
    return pl.pallas_call(
        _r_body,
        grid=(E // _RBLK,),
        in_specs=[pl.BlockSpec((_RBLK, EDGE_DIM), lambda i: (i, 0)),
                  pl.BlockSpec((EDGE_DIM, EMB), lambda i: (0, 0)),
                  pl.BlockSpec((1, EMB), lambda i: (0, 0))],
        out_specs=pl.BlockSpec((_RBLK, EMB), lambda i: (i, 0)),
        out_shape=jax.ShapeDtypeStruct((E, EMB), _F32),
    )(ea, w, b.reshape(1, EMB))


_DBLK = 2000


def _edge2_body(h1_ref, s_ref, t_ref, w_ref, b_ref, a2_ref, st_ref, acc_ref):
    i = pl.program_id(0)

    @pl.when(i == 0)
    def _():
        acc_ref[...] = jnp.zeros_like(acc_ref)

    z = jnp.maximum(h1_ref[...] * s_ref[...] + t_ref[...], 0.0)
    a2 = _dot(z, w_ref[...]) + b_ref[...]
    a2_ref[...] = a2
    a2r = a2.reshape(_DBLK // 8, 8, EMB)
    acc_ref[0] += jnp.sum(a2r, axis=0)
    acc_ref[1] += jnp.sum(a2r * a2r, axis=0)

    @pl.when(i == pl.num_programs(0) - 1)
    def _():
        st_ref[...] = acc_ref[...]


def _edge2(h1, s, t, w, b):
    return pl.pallas_call(
        _edge2_body,
        grid=(E // _DBLK,),
        in_specs=[pl.BlockSpec((_DBLK, EMB), lambda i: (i, 0)),
                  pl.BlockSpec((1, EMB), lambda i: (0, 0)),
                  pl.BlockSpec((1, EMB), lambda i: (0, 0)),
                  pl.BlockSpec((EMB, EMB), lambda i: (0, 0)),
                  pl.BlockSpec((1, EMB), lambda i: (0, 0))],
        out_specs=[pl.BlockSpec((_DBLK, EMB), lambda i: (i, 0)),
                   pl.BlockSpec((2, 8, EMB), lambda i: (0, 0, 0))],
        out_shape=[jax.ShapeDtypeStruct((E, EMB), _F32),
                   jax.ShapeDtypeStruct((2, 8, EMB), _F32)],
        scratch_shapes=[pltpu.VMEM((2, 8, EMB), _F32)],
    )(h1, s.reshape(1, EMB), t.reshape(1, EMB), w, b.reshape(1, EMB))


def _upd_body(h_ref, parts_ref, uh_ref, ua_ref, bu_ref, g1_ref, c1_ref,
              w2_ref, b2_ref, g2_ref, c2_ref, o_ref):
    h = h_ref[...]
    agg = parts_ref[0, :N, :EMB] + parts_ref[1, :N, :EMB]
    u = _dot(h, uh_ref[...]) + _dot(agg, ua_ref[...]) + bu_ref[...]
    m = jnp.mean(u, axis=0, keepdims=True)
    v = jnp.mean(u * u, axis=0, keepdims=True) - m * m
    s1 = g1_ref[...] * lax.rsqrt(v + EPS)
    z = jnp.maximum(u * s1 + (c1_ref[...] - m * s1), 0.0)
    u2 = _dot(z, w2_ref[...]) + b2_ref[...]
    m2 = jnp.mean(u2, axis=0, keepdims=True)
    v2 = jnp.mean(u2 * u2, axis=0, keepdims=True) - m2 * m2
    s2 = g2_ref[...] * lax.rsqrt(v2 + EPS)
    o_ref[...] = h + jnp.maximum(u2 * s2 + (c2_ref[...] - m2 * s2), 0.0)


def _upd(h, parts, uh, ua, bu, g1, c1, w2, b2, g2, c2):
    r1 = lambda a: a.reshape(1, EMB)
    return pl.pallas_call(
        _upd_body,
        out_shape=jax.ShapeDtypeStruct((N, EMB), _F32),
    )(h, parts, uh, ua, r1(bu), r1(g1), r1(c1), w2, r1(b2), r1(g2), r1(c2))


def _head_body(h_ref, w_ref, b_ref, o_ref):
    hm = jnp.mean(h_ref[...], axis=0, keepdims=True)
    o_ref[...] = jnp.sum(hm * w_ref[...], keepdims=True).reshape(1, 1) + b_ref[...]


def _head(h, w, b):
    return pl.pallas_call(
        _head_body,
        out_shape=jax.ShapeDtypeStruct((1, 1), _F32),
    )(h, w.reshape(1, EMB), b.reshape(1, 1))


def _sc_gather_body(t_hbm, r_hbm, d2_hbm, s2_hbm, h1_hbm, st_hbm,
                    idxd_v, idxs_v, pv, qv, rv, hv, stv, sem, sem2, sem3):
    cid = lax.axis_index("c")
    sid = lax.axis_index("s")
    wid = cid * NS + sid

    g0 = wid * NCHUNK
    zero = jnp.zeros((16,), _F32)
    init = (zero,) * 8

    def chunk(i, acc):
        g = g0 + i
        e0 = g * CHUNK
        pltpu.sync_copy(d2_hbm.at[g], idxd_v)
        pltpu.sync_copy(s2_hbm.at[g], idxs_v)
        cps = []
        for j in range(NSUB):
            cps.append(pltpu.async_copy(
                t_hbm.at[idxd_v.at[j]], pv.at[pl.ds(j * SUB, SUB)], sem))
            cps.append(pltpu.async_copy(
                t_hbm.at[idxs_v.at[j]], qv.at[pl.ds(j * SUB, SUB)], sem2))
        cps.append(pltpu.async_copy(r_hbm.at[pl.ds(e0, CHUNK)], rv, sem3))
        for cp in cps:
            cp.wait()

        def row(rr, a):
            sums = list(a[:4])
            sqs = list(a[4:])
            for v in range(4):
                val = (pv[rr, pl.ds(v * 16, 16)]
                       + qv[rr, pl.ds(EMB + v * 16, 16)]
                       + rv[rr, pl.ds(v * 16, 16)])
                hv[rr, pl.ds(v * 16, 16)] = val
                sums[v] = sums[v] + val
                sqs[v] = sqs[v] + val * val
            return tuple(sums) + tuple(sqs)

        acc = lax.fori_loop(0, CHUNK, row, acc)
        pltpu.sync_copy(hv, h1_hbm.at[pl.ds(e0, CHUNK)])
        return acc

    acc = lax.fori_loop(0, NCHUNK, chunk, init)
    for v in range(4):
        stv[0, pl.ds(v * 16, 16)] = acc[v]
        stv[0, pl.ds(64 + v * 16, 16)] = acc[4 + v]
    pltpu.sync_copy(stv, st_hbm.at[wid])


@functools.partial(
    pl.kernel,
    out_type=[jax.ShapeDtypeStruct((E, EMB), _F32),
              jax.ShapeDtypeStruct((NW, 8, 2 * EMB), _F32)],
    mesh=_vmesh,
    scratch_types=[pltpu.VMEM((NSUB, SUB), jnp.int32),
                   pltpu.VMEM((NSUB, SUB), jnp.int32),
                   pltpu.VMEM((CHUNK, 2 * EMB), _F32),
                   pltpu.VMEM((CHUNK, 2 * EMB), _F32),
                   pltpu.VMEM((CHUNK, EMB), _F32),
                   pltpu.VMEM((CHUNK, EMB), _F32),
                   pltpu.VMEM((8, 2 * EMB), _F32),
                   pltpu.SemaphoreType.DMA,
                   pltpu.SemaphoreType.DMA,
                   pltpu.SemaphoreType.DMA],
)
def _sc_gather(*refs):
    _sc_gather_body(*refs)


def _sc_scatter_body(a2_hbm, d2_hbm, ss_hbm, out_hbm,
                     shared, idx_v, mv, mj, ssv, zv, sem):
    cid = lax.axis_index("c")
    sid = lax.axis_index("s")
    wid = cid * NS + sid

    zero = jnp.zeros((16,), _F32)

    def zrow(rr, _):
        for v in range(8):
            zv[rr, pl.ds(v * 16, 16)] = zero
        return 0

    lax.fori_loop(0, ZROWS, zrow, 0)

    def mjz(rr, _):
        for v in range(4):
            mj[rr, pl.ds(EMB + v * 16, 16)] = zero
        return 0

    lax.fori_loop(0, SUB2, mjz, 0)
    for k in range(RPT // ZROWS):
        pltpu.sync_copy(zv, shared.at[pl.ds(sid * RPT + k * ZROWS, ZROWS)])
    pltpu.sync_copy(ss_hbm, ssv)
    plsc.subcore_barrier()

    g0 = wid * NCH2

    def chunk(i, _):
        g = g0 + i
        e0 = g * SUB2
        pltpu.sync_copy(d2_hbm.at[g], idx_v)
        pltpu.sync_copy(a2_hbm.at[pl.ds(e0, SUB2)], mv)

        def row(rr, carry):
            for v in range(4):
                sl = pl.ds(v * 16, 16)
                x = (mv[rr, sl] * ssv[pl.ds(v * 16, 16)]
                     + ssv[pl.ds(64 + v * 16, 16)])
                mj[rr, sl] = jnp.maximum(x, 0.0)
            return carry

        lax.fori_loop(0, SUB2, row, 0)
        pltpu.sync_copy(mj, shared.at[idx_v.at[0]], add=True)
        return 0

    lax.fori_loop(0, NCH2, chunk, 0)
    plsc.subcore_barrier()
    for k in range(RPT // ZROWS):
        sl = pl.ds(sid * RPT + k * ZROWS, ZROWS)
        pltpu.sync_copy(shared.at[sl], zv)
        pltpu.sync_copy(zv, out_hbm.at[cid, sl])


@functools.partial(
    pl.kernel,
    out_type=jax.ShapeDtypeStruct((NC, NPAD, 2 * EMB), _F32),
    mesh=_vmesh,
    scratch_types=[pltpu.VMEM_SHARED((NPAD, 2 * EMB), _F32),
                   pltpu.VMEM((1, SUB2), jnp.int32),
                   pltpu.VMEM((SUB2, EMB), _F32),
                   pltpu.VMEM((SUB2, 2 * EMB), _F32),
                   pltpu.VMEM((2 * EMB,), _F32),
                   pltpu.VMEM((ZROWS, 2 * EMB), _F32),
                   pltpu.SemaphoreType.DMA],
)
def _sc_scatter(*refs):
    _sc_scatter_body(*refs)


def kernel(x, edge_index, edge_attr, params):
    src = edge_index[0]
    dst = edge_index[1]
    d2 = dst.reshape(DROWS, NSUB, SUB)
    s2 = src.reshape(DROWS, NSUB, SUB)
    d2b = dst.reshape(E // SUB2, 1, SUB2)

    h = _lin(x, params["lin_in"]["W"], params["lin_in"]["b"])

    for p in params["layers"]:
        w1 = p["msg_lin1"]["W"]
        b1 = p["msg_lin1"]["b"]
        wab = jnp.concatenate([w1[:EMB], w1[EMB:2 * EMB]], axis=1)
        T = _pq(h, wab)
        R = _r(edge_attr, w1[2 * EMB:], b1)
        h1, st = _sc_gather(T, R, d2, s2)
        ssum = jnp.sum(st[:, 0, :EMB], axis=0)
        ssq = jnp.sum(st[:, 0, EMB:], axis=0)
        m1 = ssum / E
        v1 = ssq / E - m1 * m1
        sc1 = p["msg_bn1"]["gamma"] * lax.rsqrt(v1 + EPS)
        sh1 = p["msg_bn1"]["beta"] - m1 * sc1

        a2, st2 = _edge2(h1, sc1, sh1, p["msg_lin2"]["W"], p["msg_lin2"]["b"])
        st2s = jnp.sum(st2, axis=1)
        m2 = st2s[0] / E
        v2 = st2s[1] / E - m2 * m2
        sc2 = p["msg_bn2"]["gamma"] * lax.rsqrt(v2 + EPS)
        sh2 = p["msg_bn2"]["beta"] - m2 * sc2

        parts = _sc_scatter(a2, d2b, jnp.concatenate([sc2, sh2]))

        uw = p["upd_lin1"]["W"]
        h = _upd(h, parts, uw[:EMB], uw[EMB:], p["upd_lin1"]["b"],
                 p["upd_bn1"]["gamma"], p["upd_bn1"]["beta"],
                 p["upd_lin2"]["W"], p["upd_lin2"]["b"],
                 p["upd_bn2"]["gamma"], p["upd_bn2"]["beta"])

    out = _head(h, params["lin_pred"]["W"], params["lin_pred"]["b"])
    return out.reshape(-1)


# speedup vs baseline: 1.8995x; 1.0457x over previous
import functools

import jax
import jax.numpy as jnp
from jax import lax
from jax.experimental import pallas as pl
from jax.experimental.pallas import tpu as pltpu
from jax.experimental.pallas import tpu_sc as plsc

N = 10000
E = 320000
EMB = 64
EDGE_DIM = 4

NC = 2
NS = 16
NW = NC * NS
EPW = E // NW
SUB = 100
NSUB = 2
CHUNK = SUB * NSUB
NCHUNK = EPW // CHUNK
DROWS = E // CHUNK
SUB2 = 80
NCH2 = EPW // SUB2
NPAD = 10240
RPT = NPAD // NS
ZROWS = 128
EPS = 1e-5

_F32 = jnp.float32
_DOT = dict(preferred_element_type=jnp.float32, precision=lax.Precision.HIGHEST)

_vmesh = plsc.VectorSubcoreMesh(core_axis_name="c", subcore_axis_name="s")


def _dot(a, b):
    return lax.dot_general(a, b, (((1,), (0,)), ((), ())), **_DOT)


def _lin_body(x_ref, w_ref, b_ref, o_ref):
    o_ref[...] = _dot(x_ref[...], w_ref[...]) + b_ref[...]


def _lin(x, w, b):
    n = x.shape[0]
    fout = w.shape[1]
    return pl.pallas_call(
        _lin_body,
        out_shape=jax.ShapeDtypeStruct((n, fout), _F32),
    )(x, w, b.reshape(1, fout))


def _pq_body(h_ref, wab_ref, t_ref):
    h = h_ref[...]
    t_ref[0:N, :] = _dot(h, wab_ref[...])
    t_ref[N:NPAD, :] = jnp.zeros((NPAD - N, 2 * EMB), _F32)


def _pq(h, wab):
    return pl.pallas_call(
        _pq_body,
        out_shape=jax.ShapeDtypeStruct((NPAD, 2 * EMB), _F32),
    )(h, wab)


_RBLK = 4000


def _r_body(ea_ref, w_ref, b_ref, o_ref):
    ea = ea_ref[...]
    acc = b_ref[...]
    for k in range(EDGE_DIM):
        acc = acc + ea[:, k:k + 1] * w_ref[k:k + 1, :]
    o_ref[...] = acc


def _r(ea, w, b):
    return pl.pallas_call(
        _r_body,
        grid=(E // _RBLK,),
        in_specs=[pl.BlockSpec((_RBLK, EDGE_DIM), lambda i: (i, 0)),
                  pl.BlockSpec((EDGE_DIM, EMB), lambda i: (0, 0)),
                  pl.BlockSpec((1, EMB), lambda i: (0, 0))],
        out_specs=pl.BlockSpec((_RBLK, EMB), lambda i: (i, 0)),
        out_shape=jax.ShapeDtypeStruct((E, EMB), _F32),
    )(ea, w, b.reshape(1, EMB))


_DBLK = 2000


def _edge2_body(h1_ref, s_ref, t_ref, w_ref, b_ref, a2_ref, st_ref, acc_ref):
    i = pl.program_id(0)

    @pl.when(i == 0)
    def _():
        acc_ref[...] = jnp.zeros_like(acc_ref)

    z = jnp.maximum(h1_ref[...] * s_ref[...] + t_ref[...], 0.0)
    a2 = _dot(z, w_ref[...]) + b_ref[...]
    a2_ref[...] = a2
    a2r = a2.reshape(_DBLK // 8, 8, EMB)
    acc_ref[0] += jnp.sum(a2r, axis=0)
    acc_ref[1] += jnp.sum(a2r * a2r, axis=0)

    @pl.when(i == pl.num_programs(0) - 1)
    def _():
        st_ref[...] = acc_ref[...]


def _edge2(h1, s, t, w, b):
    return pl.pallas_call(
        _edge2_body,
        grid=(E // _DBLK,),
        in_specs=[pl.BlockSpec((_DBLK, EMB), lambda i: (i, 0)),
                  pl.BlockSpec((1, EMB), lambda i: (0, 0)),
                  pl.BlockSpec((1, EMB), lambda i: (0, 0)),
                  pl.BlockSpec((EMB, EMB), lambda i: (0, 0)),
                  pl.BlockSpec((1, EMB), lambda i: (0, 0))],
        out_specs=[pl.BlockSpec((_DBLK, EMB), lambda i: (i, 0)),
                   pl.BlockSpec((2, 8, EMB), lambda i: (0, 0, 0))],
        out_shape=[jax.ShapeDtypeStruct((E, EMB), _F32),
                   jax.ShapeDtypeStruct((2, 8, EMB), _F32)],
        scratch_shapes=[pltpu.VMEM((2, 8, EMB), _F32)],
    )(h1, s.reshape(1, EMB), t.reshape(1, EMB), w, b.reshape(1, EMB))


def _upd_body(h_ref, parts_ref, uh_ref, ua_ref, bu_ref, g1_ref, c1_ref,
              w2_ref, b2_ref, g2_ref, c2_ref, o_ref):
    h = h_ref[...]
    agg = parts_ref[0, :N, :EMB] + parts_ref[1, :N, :EMB]
    u = _dot(h, uh_ref[...]) + _dot(agg, ua_ref[...]) + bu_ref[...]
    m = jnp.mean(u, axis=0, keepdims=True)
    v = jnp.mean(u * u, axis=0, keepdims=True) - m * m
    s1 = g1_ref[...] * lax.rsqrt(v + EPS)
    z = jnp.maximum(u * s1 + (c1_ref[...] - m * s1), 0.0)
    u2 = _dot(z, w2_ref[...]) + b2_ref[...]
    m2 = jnp.mean(u2, axis=0, keepdims=True)
    v2 = jnp.mean(u2 * u2, axis=0, keepdims=True) - m2 * m2
    s2 = g2_ref[...] * lax.rsqrt(v2 + EPS)
    o_ref[...] = h + jnp.maximum(u2 * s2 + (c2_ref[...] - m2 * s2), 0.0)


def _upd(h, parts, uh, ua, bu, g1, c1, w2, b2, g2, c2):
    r1 = lambda a: a.reshape(1, EMB)
    return pl.pallas_call(
        _upd_body,
        out_shape=jax.ShapeDtypeStruct((N, EMB), _F32),
    )(h, parts, uh, ua, r1(bu), r1(g1), r1(c1), w2, r1(b2), r1(g2), r1(c2))


def _head_body(h_ref, w_ref, b_ref, o_ref):
    hm = jnp.mean(h_ref[...], axis=0, keepdims=True)
    o_ref[...] = jnp.sum(hm * w_ref[...], keepdims=True).reshape(1, 1) + b_ref[...]


def _head(h, w, b):
    return pl.pallas_call(
        _head_body,
        out_shape=jax.ShapeDtypeStruct((1, 1), _F32),
    )(h, w.reshape(1, EMB), b.reshape(1, 1))


def _sc_gather_body(t_hbm, r_hbm, d2_hbm, s2_hbm, h1_hbm, st_hbm,
                    idxd_v, idxs_v, pv, qv, rv, hv, stv, sem, sem2, sem3):
    cid = lax.axis_index("c")
    sid = lax.axis_index("s")
    wid = cid * NS + sid

    cpi = pltpu.async_copy(d2_hbm.at[wid], idxd_v, sem)
    cpj = pltpu.async_copy(s2_hbm.at[wid], idxs_v, sem2)
    cpi.wait()
    cpj.wait()

    e_base = wid * EPW
    zero = jnp.zeros((16,), _F32)
    init = (zero,) * 8

    def chunk(i, acc):
        e0 = e_base + i * CHUNK
        cps = []
        for j in range(NSUB):
            cps.append(pltpu.async_copy(
                t_hbm.at[idxd_v.at[i * NSUB + j]],
                pv.at[pl.ds(j * SUB, SUB)], sem))
            cps.append(pltpu.async_copy(
                t_hbm.at[idxs_v.at[i * NSUB + j]],
                qv.at[pl.ds(j * SUB, SUB)], sem2))
        cps.append(pltpu.async_copy(r_hbm.at[pl.ds(e0, CHUNK)], rv, sem3))
        for cp in cps:
            cp.wait()

        def row(rr, a):
            sums = list(a[:4])
            sqs = list(a[4:])
            for v in range(4):
                val = (pv[rr, pl.ds(v * 16, 16)]
                       + qv[rr, pl.ds(EMB + v * 16, 16)]
                       + rv[rr, pl.ds(v * 16, 16)])
                hv[rr, pl.ds(v * 16, 16)] = val
                sums[v] = sums[v] + val
                sqs[v] = sqs[v] + val * val
            return tuple(sums) + tuple(sqs)

        acc = lax.fori_loop(0, CHUNK, row, acc)
        pltpu.sync_copy(hv, h1_hbm.at[pl.ds(e0, CHUNK)])
        return acc

    acc = lax.fori_loop(0, NCHUNK, chunk, init)
    for v in range(4):
        stv[0, pl.ds(v * 16, 16)] = acc[v]
        stv[0, pl.ds(64 + v * 16, 16)] = acc[4 + v]
    pltpu.sync_copy(stv, st_hbm.at[wid])


@functools.partial(
    pl.kernel,
    out_type=[jax.ShapeDtypeStruct((E, EMB), _F32),
              jax.ShapeDtypeStruct((NW, 8, 2 * EMB), _F32)],
    mesh=_vmesh,
    scratch_types=[pltpu.VMEM((NCHUNK * NSUB, SUB), jnp.int32),
                   pltpu.VMEM((NCHUNK * NSUB, SUB), jnp.int32),
                   pltpu.VMEM((CHUNK, 2 * EMB), _F32),
                   pltpu.VMEM((CHUNK, 2 * EMB), _F32),
                   pltpu.VMEM((CHUNK, EMB), _F32),
                   pltpu.VMEM((CHUNK, EMB), _F32),
                   pltpu.VMEM((8, 2 * EMB), _F32),
                   pltpu.SemaphoreType.DMA,
                   pltpu.SemaphoreType.DMA,
                   pltpu.SemaphoreType.DMA],
)
def _sc_gather(*refs):
    _sc_gather_body(*refs)


def _sc_scatter_body(a2_hbm, d2_hbm, ss_hbm, out_hbm,
                     shared, idx_v, mv, mj, ssv, zv, sem):
    cid = lax.axis_index("c")
    sid = lax.axis_index("s")
    wid = cid * NS + sid

    zero = jnp.zeros((16,), _F32)

    def zrow(rr, _):
        for v in range(8):
            zv[rr, pl.ds(v * 16, 16)] = zero
        return 0

    lax.fori_loop(0, ZROWS, zrow, 0)

    def mjz(rr, _):
        for v in range(4):
            mj[rr, pl.ds(EMB + v * 16, 16)] = zero
        return 0

    lax.fori_loop(0, SUB2, mjz, 0)
    for k in range(RPT // ZROWS):
        pltpu.sync_copy(zv, shared.at[pl.ds(sid * RPT + k * ZROWS, ZROWS)])
    pltpu.sync_copy(ss_hbm, ssv)
    plsc.subcore_barrier()

    g0 = wid * NCH2

    def chunk(i, _):
        g = g0 + i
        e0 = g * SUB2
        pltpu.sync_copy(d2_hbm.at[g], idx_v)
        pltpu.sync_copy(a2_hbm.at[pl.ds(e0, SUB2)], mv)

        def row(rr, carry):
            for v in range(4):
                sl = pl.ds(v * 16, 16)
                x = (mv[rr, sl] * ssv[pl.ds(v * 16, 16)]
                     + ssv[pl.ds(64 + v * 16, 16)])
                mj[rr, sl] = jnp.maximum(x, 0.0)
            return carry

        lax.fori_loop(0, SUB2, row, 0)
        pltpu.sync_copy(mj, shared.at[idx_v.at[0]], add=True)
        return 0

    lax.fori_loop(0, NCH2, chunk, 0)
    plsc.subcore_barrier()
    for k in range(RPT // ZROWS):
        sl = pl.ds(sid * RPT + k * ZROWS, ZROWS)
        pltpu.sync_copy(shared.at[sl], zv)
        pltpu.sync_copy(zv, out_hbm.at[cid, sl])


@functools.partial(
    pl.kernel,
    out_type=jax.ShapeDtypeStruct((NC, NPAD, 2 * EMB), _F32),
    mesh=_vmesh,
    scratch_types=[pltpu.VMEM_SHARED((NPAD, 2 * EMB), _F32),
                   pltpu.VMEM((1, SUB2), jnp.int32),
                   pltpu.VMEM((SUB2, EMB), _F32),
                   pltpu.VMEM((SUB2, 2 * EMB), _F32),
                   pltpu.VMEM((2 * EMB,), _F32),
                   pltpu.VMEM((ZROWS, 2 * EMB), _F32),
                   pltpu.SemaphoreType.DMA],
)
def _sc_scatter(*refs):
    _sc_scatter_body(*refs)


def kernel(x, edge_index, edge_attr, params):
    src = edge_index[0]
    dst = edge_index[1]
    d2 = dst.reshape(NW, NCHUNK * NSUB, SUB)
    s2 = src.reshape(NW, NCHUNK * NSUB, SUB)
    d2b = dst.reshape(E // SUB2, 1, SUB2)

    h = _lin(x, params["lin_in"]["W"], params["lin_in"]["b"])

    for p in params["layers"]:
        w1 = p["msg_lin1"]["W"]
        b1 = p["msg_lin1"]["b"]
        wab = jnp.concatenate([w1[:EMB], w1[EMB:2 * EMB]], axis=1)
        T = _pq(h, wab)
        R = _r(edge_attr, w1[2 * EMB:], b1)
        h1, st = _sc_gather(T, R, d2, s2)
        ssum = jnp.sum(st[:, 0, :EMB], axis=0)
        ssq = jnp.sum(st[:, 0, EMB:], axis=0)
        m1 = ssum / E
        v1 = ssq / E - m1 * m1
        sc1 = p["msg_bn1"]["gamma"] * lax.rsqrt(v1 + EPS)
        sh1 = p["msg_bn1"]["beta"] - m1 * sc1

        a2, st2 = _edge2(h1, sc1, sh1, p["msg_lin2"]["W"], p["msg_lin2"]["b"])
        st2s = jnp.sum(st2, axis=1)
        m2 = st2s[0] / E
        v2 = st2s[1] / E - m2 * m2
        sc2 = p["msg_bn2"]["gamma"] * lax.rsqrt(v2 + EPS)
        sh2 = p["msg_bn2"]["beta"] - m2 * sc2

        parts = _sc_scatter(a2, d2b, jnp.concatenate([sc2, sh2]))

        uw = p["upd_lin1"]["W"]
        h = _upd(h, parts, uw[:EMB], uw[EMB:], p["upd_lin1"]["b"],
                 p["upd_bn1"]["gamma"], p["upd_bn1"]["beta"],
                 p["upd_lin2"]["W"], p["upd_lin2"]["b"],
                 p["upd_bn2"]["gamma"], p["upd_bn2"]["beta"])

    out = _head(h, params["lin_pred"]["W"], params["lin_pred"]["b"])
    return out.reshape(-1)
